# Initial kernel scaffold; baseline (speedup 1.0000x reference)
import jax
import jax.numpy as jnp
from jax.experimental import pallas as pl


def kernel(attention):
    raise NotImplementedError("write your pallas kernel here")


# speedup vs baseline: 12.2184x; 12.2184x over previous
import functools

import jax
import jax.numpy as jnp
import numpy as np
from jax import lax
from jax.experimental import pallas as pl
from jax.experimental.pallas import tpu as pltpu
from jax.experimental.pallas import tpu_sc as plsc

_BS = 16384
_F = 196
_C = 49
_KEEP = 25
_NW = 32
_RPW = _BS // _NW
_CHUNK = 128
_NCHUNK = _RPW // _CHUNK
_NG = _CHUNK // 16

_W7 = np.zeros((7, 14), np.float64)
_W7[0, 0:3] = [3 / 7, 3 / 7, 1 / 7]
for _i in range(1, 6):
    _W7[_i, 2 * _i - 1:2 * _i + 3] = [0.125, 0.375, 0.375, 0.125]
_W7[6, 11:14] = [1 / 7, 3 / 7, 3 / 7]


def _word_layout():
    words = []
    for w in range(_C):
        mult = {}
        for k in range(4):
            f = 4 * w + k
            r, c = divmod(f, 14)
            cell = 7 * (r // 2) + c // 2
            mult[cell] = mult.get(cell, 0) + (1 << (8 * k))
        words.append(sorted(mult.items()))
    return words


_WORDS = _word_layout()


def _sc_mask_words(att_flat):
    mesh = plsc.VectorSubcoreMesh(core_axis_name="c", subcore_axis_name="s")

    @functools.partial(
        pl.kernel,
        mesh=mesh,
        out_type=jax.ShapeDtypeStruct((_BS * _C,), jnp.int32),
        compiler_params=pltpu.CompilerParams(needs_layout_passes=False),
        scratch_types=[
            pltpu.VMEM((_CHUNK * _F,), jnp.float32),
            pltpu.VMEM((_CHUNK * _C,), jnp.int32),
            pltpu.VMEM((98 * 16,), jnp.float32),
            pltpu.VMEM((_C * 16,), jnp.int32),
            pltpu.VMEM((_C * 16,), jnp.int32),
        ],
    )
    def body(att_hbm, out_hbm, in_v, out_v, hbuf, kbuf, mbuf):
        nc = 2
        wid = lax.axis_index("s") * nc + lax.axis_index("c")
        iota = lax.iota(jnp.int32, 16)

        def chunk_body(ch, carry):
            base = wid * _RPW + ch * _CHUNK
            pltpu.sync_copy(att_hbm.at[pl.ds(base * _F, _CHUNK * _F)], in_v)

            def group_body(g, carry2):
                riota = g * 16 + iota
                gbase = riota * _F
                for r in range(14):
                    cols = [
                        plsc.load_gather(in_v, [gbase + (r * 14 + x)])
                        for x in range(14)
                    ]
                    for j in range(7):
                        acc = None
                        for x in range(14):
                            wgt = float(_W7[j, x])
                            if wgt != 0.0:
                                t = cols[x] * jnp.float32(wgt)
                                acc = t if acc is None else acc + t
                        hbuf[pl.ds((j * 14 + r) * 16, 16)] = acc
                for j in range(7):
                    hs = [
                        hbuf[pl.ds((j * 14 + r) * 16, 16)] for r in range(14)
                    ]
                    for i in range(7):
                        acc = None
                        for r in range(14):
                            wgt = float(_W7[i, r])
                            if wgt != 0.0:
                                t = hs[r] * jnp.float32(wgt)
                                acc = t if acc is None else acc + t
                        u = lax.bitcast_convert_type(acc, jnp.int32)
                        kbuf[pl.ds((i * 7 + j) * 16, 16)] = u ^ (
                            (u >> 31) & jnp.int32(0x7FFFFFFF)
                        )

                def tile_body(t, carry3):
                    p0 = t * 7
                    kp = [
                        kbuf[pl.ds((p0 + jj) * 16, 16)] for jj in range(7)
                    ]
                    accs = [jnp.zeros((16,), jnp.int32) for _ in range(7)]
                    for q in range(_C):
                        kq = kbuf[pl.ds(q * 16, 16)]
                        for jj in range(7):
                            accs[jj] = accs[jj] + (kq > kp[jj]).astype(
                                jnp.int32
                            )
                    for jj in range(7):
                        mbuf[pl.ds((p0 + jj) * 16, 16)] = (
                            accs[jj] >= _KEEP
                        ).astype(jnp.int32)
                    return carry3

                lax.fori_loop(0, 7, tile_body, 0)

                obase = riota * _C
                for w, cells in enumerate(_WORDS):
                    word = None
                    for cell, mult in cells:
                        t = mbuf[pl.ds(cell * 16, 16)] * jnp.int32(mult)
                        word = t if word is None else word + t
                    plsc.store_scatter(out_v, [obase + w], word)
                return carry2

            lax.fori_loop(0, _NG, group_body, 0)
            pltpu.sync_copy(out_v, out_hbm.at[pl.ds(base * _C, _CHUNK * _C)])
            return carry

        lax.fori_loop(0, _NCHUNK, chunk_body, 0)

    return body(att_flat)


def kernel(attention):
    assert attention.shape == (_BS, _F) and attention.dtype == jnp.float32
    words = _sc_mask_words(attention.reshape(_BS * _F))
    out_bytes = lax.bitcast_convert_type(words, jnp.int8)
    return out_bytes.reshape(_BS, _F) != 0


# speedup vs baseline: 16.7925x; 1.3744x over previous
import functools

import jax
import jax.numpy as jnp
import numpy as np
from jax import lax
from jax.experimental import pallas as pl
from jax.experimental.pallas import tpu as pltpu
from jax.experimental.pallas import tpu_sc as plsc

_BS = 16384
_F = 196
_C = 49
_NW = 32
_RPW = _BS // _NW
_CHUNK = 128
_NCHUNK = _RPW // _CHUNK
_NG = _CHUNK // 16

_B0 = np.float32(24.0 / 7.0)
_B1 = np.float32(8.0 / 7.0)
_THREE = np.float32(3.0)


def _word_layout():
    words = []
    for w in range(_C):
        mult = {}
        for k in range(4):
            f = 4 * w + k
            r, c = divmod(f, 14)
            cell = 7 * (r // 2) + c // 2
            mult[cell] = mult.get(cell, 0) + (1 << (8 * k))
        words.append(sorted(mult.items()))
    return words


_WORDS = _word_layout()


def _build_network(n_real=49, n=64, out_pos=39):
    const = -1
    wires = list(range(n_real)) + [const] * (n - n_real)
    next_id = n_real
    ops = []
    k = 2
    while k <= n:
        j = k // 2
        while j >= 1:
            for i in range(n):
                l = i ^ j
                if l > i:
                    asc = (i & k) == 0
                    a, b = wires[i], wires[l]
                    if a == const and b == const:
                        continue
                    if a == const or b == const:
                        other = b if a == const else a
                        wires[i], wires[l] = (
                            (const, other) if asc else (other, const)
                        )
                        continue
                    mn, mx = next_id, next_id + 1
                    next_id += 2
                    ops.append((a, b, mn, mx))
                    wires[i], wires[l] = (mn, mx) if asc else (mx, mn)
            j //= 2
        k *= 2
    out_id = wires[out_pos]
    needed = {out_id}
    kept = []
    for a, b, mn, mx in reversed(ops):
        nmn, nmx = mn in needed, mx in needed
        if nmn or nmx:
            kept.append((a, b, mn if nmn else None, mx if nmx else None))
            needed.add(a)
            needed.add(b)
    kept.reverse()
    return kept, out_id


_NET_OPS, _NET_OUT = _build_network()


def _select_threshold(vals):
    env = {i: vals[i] for i in range(_C)}
    for a, b, mn, mx in _NET_OPS:
        if mn is not None:
            env[mn] = jnp.minimum(env[a], env[b])
        if mx is not None:
            env[mx] = jnp.maximum(env[a], env[b])
    return env[_NET_OUT]


def _down7(vals):
    out = [None] * 7
    out[0] = vals[0] * _B0 + (vals[1] * _B0 + vals[2] * _B1)
    for j in range(1, 6):
        a, b, c, d = vals[2 * j - 1:2 * j + 3]
        out[j] = (a + d) + (b + c) * _THREE
    out[6] = vals[13] * _B0 + (vals[12] * _B0 + vals[11] * _B1)
    return out


def _sc_mask_words(att_flat):
    mesh = plsc.VectorSubcoreMesh(core_axis_name="c", subcore_axis_name="s")

    @functools.partial(
        pl.kernel,
        mesh=mesh,
        out_type=jax.ShapeDtypeStruct((_BS * _C,), jnp.int32),
        compiler_params=pltpu.CompilerParams(needs_layout_passes=False),
        scratch_types=[
            pltpu.VMEM((_CHUNK * _F,), jnp.float32),
            pltpu.VMEM((_CHUNK * _C,), jnp.int32),
        ],
    )
    def body(att_hbm, out_hbm, in_v, out_v):
        nc = 2
        wid = lax.axis_index("s") * nc + lax.axis_index("c")
        iota = lax.iota(jnp.int32, 16)

        def chunk_body(ch, carry):
            base = wid * _RPW + ch * _CHUNK
            pltpu.sync_copy(att_hbm.at[pl.ds(base * _F, _CHUNK * _F)], in_v)

            def group_body(g, carry2):
                riota = g * 16 + iota
                gbase = riota * _F
                h = {}
                v = [None] * _C
                for r in range(14):
                    cols = [
                        plsc.load_gather(in_v, [gbase + (r * 14 + x)])
                        for x in range(14)
                    ]
                    h[r] = _down7(cols)
                    if r == 2:
                        for j in range(7):
                            v[j] = h[0][j] * _B0 + (
                                h[1][j] * _B0 + h[2][j] * _B1
                            )
                    if r >= 4 and r % 2 == 0:
                        i = r // 2 - 1
                        if i <= 5:
                            for j in range(7):
                                a = h[2 * i - 1][j]
                                b = h[2 * i][j]
                                c = h[2 * i + 1][j]
                                d = h[2 * i + 2][j]
                                v[i * 7 + j] = (a + d) + (b + c) * _THREE
                    if r == 13:
                        for j in range(7):
                            v[6 * 7 + j] = h[13][j] * _B0 + (
                                h[12][j] * _B0 + h[11][j] * _B1
                            )

                thr = _select_threshold(v)
                m = [(v[p] < thr).astype(jnp.int32) for p in range(_C)]

                obase = riota * _C
                for w, cells in enumerate(_WORDS):
                    word = None
                    for cell, mult in cells:
                        t = m[cell] * jnp.int32(mult)
                        word = t if word is None else word + t
                    plsc.store_scatter(out_v, [obase + w], word)
                return carry2

            lax.fori_loop(0, _NG, group_body, 0)
            pltpu.sync_copy(out_v, out_hbm.at[pl.ds(base * _C, _CHUNK * _C)])
            return carry

        lax.fori_loop(0, _NCHUNK, chunk_body, 0)

    return body(att_flat)


def kernel(attention):
    assert attention.shape == (_BS, _F) and attention.dtype == jnp.float32
    words = _sc_mask_words(attention.reshape(_BS * _F))
    out_bytes = lax.bitcast_convert_type(words, jnp.int8)
    return out_bytes.reshape(_BS, _F) != 0


# speedup vs baseline: 19.7993x; 1.1791x over previous
import functools

import jax
import jax.numpy as jnp
import numpy as np
from jax import lax
from jax.experimental import pallas as pl
from jax.experimental.pallas import tpu as pltpu
from jax.experimental.pallas import tpu_sc as plsc

_BS = 16384
_F = 196
_C = 49
_NW = 32
_RPW = _BS // _NW
_CHUNK = 128
_NCHUNK = _RPW // _CHUNK
_NG = _CHUNK // 64

_B0 = np.float32(24.0 / 7.0)
_B1 = np.float32(8.0 / 7.0)
_THREE = np.float32(3.0)

_CELL = [7 * ((f // 14) // 2) + (f % 14) // 2 for f in range(_F)]


def _build_network(n_real=49, n=64, out_pos=39):
    const = -1
    wires = list(range(n_real)) + [const] * (n - n_real)
    next_id = n_real
    ops = []
    k = 2
    while k <= n:
        j = k // 2
        while j >= 1:
            for i in range(n):
                l = i ^ j
                if l > i:
                    asc = (i & k) == 0
                    a, b = wires[i], wires[l]
                    if a == const and b == const:
                        continue
                    if a == const or b == const:
                        other = b if a == const else a
                        wires[i], wires[l] = (
                            (const, other) if asc else (other, const)
                        )
                        continue
                    mn, mx = next_id, next_id + 1
                    next_id += 2
                    ops.append((a, b, mn, mx))
                    wires[i], wires[l] = (mn, mx) if asc else (mx, mn)
            j //= 2
        k *= 2
    out_id = wires[out_pos]
    needed = {out_id}
    kept = []
    for a, b, mn, mx in reversed(ops):
        nmn, nmx = mn in needed, mx in needed
        if nmn or nmx:
            kept.append((a, b, mn if nmn else None, mx if nmx else None))
            needed.add(a)
            needed.add(b)
    kept.reverse()
    return kept, out_id


_NET_OPS, _NET_OUT = _build_network()


def _select_threshold(vals):
    env = {i: vals[i] for i in range(_C)}
    for a, b, mn, mx in _NET_OPS:
        if mn is not None:
            env[mn] = jnp.minimum(env[a], env[b])
        if mx is not None:
            env[mx] = jnp.maximum(env[a], env[b])
    return env[_NET_OUT]


def _down7(vals):
    out = [None] * 7
    out[0] = vals[0] * _B0 + (vals[1] * _B0 + vals[2] * _B1)
    for j in range(1, 6):
        a, b, c, d = vals[2 * j - 1:2 * j + 3]
        out[j] = (a + d) + (b + c) * _THREE
    out[6] = vals[13] * _B0 + (vals[12] * _B0 + vals[11] * _B1)
    return out


def _sc_mask_bytes(att):
    mesh = plsc.VectorSubcoreMesh(core_axis_name="c", subcore_axis_name="s")

    @functools.partial(
        pl.kernel,
        mesh=mesh,
        out_type=jax.ShapeDtypeStruct((_BS, _F), jnp.int8),
        compiler_params=pltpu.CompilerParams(needs_layout_passes=False),
        scratch_types=[
            pltpu.VMEM((_CHUNK, _F), jnp.float32),
            pltpu.VMEM((_CHUNK // 4, _F), jnp.int32),
            pltpu.VMEM((4 * _C * 16,), jnp.int32),
        ],
    )
    def body(att_hbm, out_i8_hbm, in_v, out_v, mbuf):
        out_hbm = out_i8_hbm.bitcast(jnp.int32)
        nc = 2
        wid = lax.axis_index("s") * nc + lax.axis_index("c")
        iota = lax.iota(jnp.int32, 16)

        def chunk_body(ch, carry):
            base = wid * _RPW + ch * _CHUNK
            pltpu.sync_copy(att_hbm.at[pl.ds(base, _CHUNK)], in_v)

            def group_body(g, carry2):
                def pass_body(p, carry3):
                    riota = g * 64 + iota * 4 + p
                    h = {}
                    v = [None] * _C
                    for r in range(14):
                        cols = [
                            plsc.load_gather(
                                in_v,
                                [riota, jnp.full((16,), r * 14 + x, jnp.int32)],
                            )
                            for x in range(14)
                        ]
                        h[r] = _down7(cols)
                        if r == 2:
                            for j in range(7):
                                v[j] = h[0][j] * _B0 + (
                                    h[1][j] * _B0 + h[2][j] * _B1
                                )
                        if r >= 4 and r % 2 == 0:
                            i = r // 2 - 1
                            if i <= 5:
                                for j in range(7):
                                    a = h[2 * i - 1][j]
                                    b = h[2 * i][j]
                                    c = h[2 * i + 1][j]
                                    d = h[2 * i + 2][j]
                                    v[i * 7 + j] = (a + d) + (b + c) * _THREE
                        if r == 13:
                            for j in range(7):
                                v[6 * 7 + j] = h[13][j] * _B0 + (
                                    h[12][j] * _B0 + h[11][j] * _B1
                                )
                    thr = _select_threshold(v)
                    mb = p * (_C * 16)
                    for q in range(_C):
                        mbuf[pl.ds(mb + q * 16, 16)] = (
                            v[q] < thr
                        ).astype(jnp.int32)
                    return carry3

                lax.fori_loop(0, 4, pass_body, 0)

                comb = []
                for q in range(_C):
                    w0 = mbuf[pl.ds(0 * (_C * 16) + q * 16, 16)]
                    w1 = mbuf[pl.ds(1 * (_C * 16) + q * 16, 16)]
                    w2 = mbuf[pl.ds(2 * (_C * 16) + q * 16, 16)]
                    w3 = mbuf[pl.ds(3 * (_C * 16) + q * 16, 16)]
                    comb.append(
                        w0 + w1 * jnp.int32(1 << 8)
                        + w2 * jnp.int32(1 << 16) + w3 * jnp.int32(1 << 24)
                    )
                wrow = g * 16 + iota
                for f in range(_F):
                    plsc.store_scatter(
                        out_v,
                        [wrow, jnp.full((16,), f, jnp.int32)],
                        comb[_CELL[f]],
                    )
                return carry2

            lax.fori_loop(0, _NG, group_body, 0)
            obase = pl.multiple_of(base // 4, _CHUNK // 4)
            pltpu.sync_copy(out_v, out_hbm.at[pl.ds(obase, _CHUNK // 4)])
            return carry

        lax.fori_loop(0, _NCHUNK, chunk_body, 0)

    return body(att)


def kernel(attention):
    assert attention.shape == (_BS, _F) and attention.dtype == jnp.float32
    return _sc_mask_bytes(attention) != 0


# speedup vs baseline: 20.2311x; 1.0218x over previous
import functools

import jax
import jax.numpy as jnp
import numpy as np
from jax import lax
from jax.experimental import pallas as pl
from jax.experimental.pallas import tpu as pltpu
from jax.experimental.pallas import tpu_sc as plsc

_BS = 16384
_F = 196
_C = 49
_NW = 32
_RPW = _BS // _NW
_CHUNK = 128
_NCHUNK = _RPW // _CHUNK
_NG = _CHUNK // 64

_B0 = np.float32(24.0 / 7.0)
_B1 = np.float32(8.0 / 7.0)
_THREE = np.float32(3.0)

_CELL = [7 * ((f // 14) // 2) + (f % 14) // 2 for f in range(_F)]


def _build_network(n_real=49, n=64, out_pos=39):
    const = -1
    wires = list(range(n_real)) + [const] * (n - n_real)
    next_id = n_real
    ops = []
    k = 2
    while k <= n:
        j = k // 2
        while j >= 1:
            for i in range(n):
                l = i ^ j
                if l > i:
                    asc = (i & k) == 0
                    a, b = wires[i], wires[l]
                    if a == const and b == const:
                        continue
                    if a == const or b == const:
                        other = b if a == const else a
                        wires[i], wires[l] = (
                            (const, other) if asc else (other, const)
                        )
                        continue
                    mn, mx = next_id, next_id + 1
                    next_id += 2
                    ops.append((a, b, mn, mx))
                    wires[i], wires[l] = (mn, mx) if asc else (mx, mn)
            j //= 2
        k *= 2
    out_id = wires[out_pos]
    needed = {out_id}
    kept = []
    for a, b, mn, mx in reversed(ops):
        nmn, nmx = mn in needed, mx in needed
        if nmn or nmx:
            kept.append((a, b, mn if nmn else None, mx if nmx else None))
            needed.add(a)
            needed.add(b)
    kept.reverse()
    return kept, out_id


_NET_OPS, _NET_OUT = _build_network()


def _select_threshold(vals):
    env = {i: vals[i] for i in range(_C)}
    for a, b, mn, mx in _NET_OPS:
        if mn is not None:
            env[mn] = jnp.minimum(env[a], env[b])
        if mx is not None:
            env[mx] = jnp.maximum(env[a], env[b])
    return env[_NET_OUT]


def _down7(vals):
    out = [None] * 7
    out[0] = vals[0] * _B0 + (vals[1] * _B0 + vals[2] * _B1)
    for j in range(1, 6):
        a, b, c, d = vals[2 * j - 1:2 * j + 3]
        out[j] = (a + d) + (b + c) * _THREE
    out[6] = vals[13] * _B0 + (vals[12] * _B0 + vals[11] * _B1)
    return out


def _sc_mask_bytes(att):
    mesh = plsc.VectorSubcoreMesh(core_axis_name="c", subcore_axis_name="s")

    @functools.partial(
        pl.kernel,
        mesh=mesh,
        out_type=jax.ShapeDtypeStruct((_BS, _F), jnp.int8),
        compiler_params=pltpu.CompilerParams(needs_layout_passes=False),
        scratch_types=[
            pltpu.VMEM((_CHUNK * _F,), jnp.float32),
            pltpu.VMEM((_CHUNK // 4, _F), jnp.int32),
            pltpu.VMEM((4 * _C * 16,), jnp.int32),
        ],
    )
    def body(att_hbm, out_i8_hbm, in_v, out_v, mbuf):
        out_hbm = out_i8_hbm.bitcast(jnp.int32)
        nc = 2
        wid = lax.axis_index("s") * nc + lax.axis_index("c")
        iota = lax.iota(jnp.int32, 16)

        def chunk_body(ch, carry):
            base = wid * _RPW + ch * _CHUNK
            pltpu.sync_copy(att_hbm.at[pl.ds(base * _F, _CHUNK * _F)], in_v)

            def group_body(g, carry2):
                def pass_body(p, carry3):
                    riota = g * 64 + iota * 4 + p
                    gbase = riota * _F
                    h = {}
                    v = [None] * _C
                    for r in range(14):
                        cols = [
                            plsc.load_gather(in_v, [gbase + (r * 14 + x)])
                            for x in range(14)
                        ]
                        h[r] = _down7(cols)
                        if r == 2:
                            for j in range(7):
                                v[j] = h[0][j] * _B0 + (
                                    h[1][j] * _B0 + h[2][j] * _B1
                                )
                        if r >= 4 and r % 2 == 0:
                            i = r // 2 - 1
                            if i <= 5:
                                for j in range(7):
                                    a = h[2 * i - 1][j]
                                    b = h[2 * i][j]
                                    c = h[2 * i + 1][j]
                                    d = h[2 * i + 2][j]
                                    v[i * 7 + j] = (a + d) + (b + c) * _THREE
                        if r == 13:
                            for j in range(7):
                                v[6 * 7 + j] = h[13][j] * _B0 + (
                                    h[12][j] * _B0 + h[11][j] * _B1
                                )
                    thr = _select_threshold(v)
                    mb = p * (_C * 16)
                    for q in range(_C):
                        mbuf[pl.ds(mb + q * 16, 16)] = (
                            v[q] < thr
                        ).astype(jnp.int32)
                    return carry3

                lax.fori_loop(0, 4, pass_body, 0)

                comb = []
                for q in range(_C):
                    w0 = mbuf[pl.ds(0 * (_C * 16) + q * 16, 16)]
                    w1 = mbuf[pl.ds(1 * (_C * 16) + q * 16, 16)]
                    w2 = mbuf[pl.ds(2 * (_C * 16) + q * 16, 16)]
                    w3 = mbuf[pl.ds(3 * (_C * 16) + q * 16, 16)]
                    comb.append(
                        w0 + w1 * jnp.int32(1 << 8)
                        + w2 * jnp.int32(1 << 16) + w3 * jnp.int32(1 << 24)
                    )
                wrow = g * 16 + iota
                for f in range(_F):
                    plsc.store_scatter(
                        out_v,
                        [wrow, jnp.full((16,), f, jnp.int32)],
                        comb[_CELL[f]],
                    )
                return carry2

            lax.fori_loop(0, _NG, group_body, 0)
            obase = pl.multiple_of(base // 4, _CHUNK // 4)
            pltpu.sync_copy(out_v, out_hbm.at[pl.ds(obase, _CHUNK // 4)])
            return carry

        lax.fori_loop(0, _NCHUNK, chunk_body, 0)

    return body(att)


def kernel(attention):
    assert attention.shape == (_BS, _F) and attention.dtype == jnp.float32
    return _sc_mask_bytes(attention.reshape(_BS * _F)) != 0


# speedup vs baseline: 21.5973x; 1.0675x over previous
import functools

import jax
import jax.numpy as jnp
import numpy as np
from jax import lax
from jax.experimental import pallas as pl
from jax.experimental.pallas import tpu as pltpu
from jax.experimental.pallas import tpu_sc as plsc

_BS = 16384
_F = 196
_C = 49
_NW = 32
_RPW = _BS // _NW
_CHUNK = 128
_NCHUNK = _RPW // _CHUNK
_NG = _CHUNK // 64

_B0 = np.float32(24.0 / 7.0)
_B1 = np.float32(8.0 / 7.0)
_THREE = np.float32(3.0)

_CELL = [7 * ((f // 14) // 2) + (f % 14) // 2 for f in range(_F)]


def _build_network(n_real=49, n=64, out_pos=39):
    const = -1
    wires = list(range(n_real)) + [const] * (n - n_real)
    next_id = n_real
    ops = []
    k = 2
    while k <= n:
        j = k // 2
        while j >= 1:
            for i in range(n):
                l = i ^ j
                if l > i:
                    asc = (i & k) == 0
                    a, b = wires[i], wires[l]
                    if a == const and b == const:
                        continue
                    if a == const or b == const:
                        other = b if a == const else a
                        wires[i], wires[l] = (
                            (const, other) if asc else (other, const)
                        )
                        continue
                    mn, mx = next_id, next_id + 1
                    next_id += 2
                    ops.append((a, b, mn, mx))
                    wires[i], wires[l] = (mn, mx) if asc else (mx, mn)
            j //= 2
        k *= 2
    out_id = wires[out_pos]
    needed = {out_id}
    kept = []
    for a, b, mn, mx in reversed(ops):
        nmn, nmx = mn in needed, mx in needed
        if nmn or nmx:
            kept.append((a, b, mn if nmn else None, mx if nmx else None))
            needed.add(a)
            needed.add(b)
    kept.reverse()
    return kept, out_id


_NET_OPS, _NET_OUT = _build_network()


def _select_threshold(vals):
    env = {i: vals[i] for i in range(_C)}
    for a, b, mn, mx in _NET_OPS:
        if mn is not None:
            env[mn] = jnp.minimum(env[a], env[b])
        if mx is not None:
            env[mx] = jnp.maximum(env[a], env[b])
    return env[_NET_OUT]


def _down7(vals):
    out = [None] * 7
    out[0] = vals[0] * _B0 + (vals[1] * _B0 + vals[2] * _B1)
    for j in range(1, 6):
        a, b, c, d = vals[2 * j - 1:2 * j + 3]
        out[j] = (a + d) + (b + c) * _THREE
    out[6] = vals[13] * _B0 + (vals[12] * _B0 + vals[11] * _B1)
    return out


def _sc_mask_bytes(att):
    mesh = plsc.VectorSubcoreMesh(core_axis_name="c", subcore_axis_name="s")

    @functools.partial(
        pl.kernel,
        mesh=mesh,
        out_type=jax.ShapeDtypeStruct((_BS, _F), jnp.int8),
        compiler_params=pltpu.CompilerParams(needs_layout_passes=False),
        scratch_types=[
            pltpu.VMEM((2 * _CHUNK * _F,), jnp.float32),
            pltpu.VMEM((_CHUNK // 4, _F), jnp.int32),
            pltpu.VMEM((4 * _C * 16,), jnp.int32),
            pltpu.SemaphoreType.DMA,
        ],
    )
    def body(att_hbm, out_i8_hbm, in_v, out_v, mbuf, insem):
        out_hbm = out_i8_hbm.bitcast(jnp.int32)
        nc = 2
        wid = lax.axis_index("s") * nc + lax.axis_index("c")
        iota = lax.iota(jnp.int32, 16)

        def in_desc(ch):
            src = att_hbm.at[
                pl.ds((wid * _RPW + ch * _CHUNK) * _F, _CHUNK * _F)
            ]
            dst = in_v.at[pl.ds((ch % 2) * (_CHUNK * _F), _CHUNK * _F)]
            return pltpu.make_async_copy(src, dst, insem)

        in_desc(0).start()

        def chunk_body(ch, carry):
            base = wid * _RPW + ch * _CHUNK
            in_desc(ch).wait()

            @pl.when(ch < _NCHUNK - 1)
            def _():
                in_desc(ch + 1).start()

            vbase = (ch % 2) * (_CHUNK * _F)

            def group_body(g, carry2):
                def pass_body(p, carry3):
                    riota = g * 64 + iota * 4 + p
                    gbase = vbase + riota * _F
                    h = {}
                    v = [None] * _C
                    for r in range(14):
                        cols = [
                            plsc.load_gather(in_v, [gbase + (r * 14 + x)])
                            for x in range(14)
                        ]
                        h[r] = _down7(cols)
                        if r == 2:
                            for j in range(7):
                                v[j] = h[0][j] * _B0 + (
                                    h[1][j] * _B0 + h[2][j] * _B1
                                )
                        if r >= 4 and r % 2 == 0:
                            i = r // 2 - 1
                            if i <= 5:
                                for j in range(7):
                                    a = h[2 * i - 1][j]
                                    b = h[2 * i][j]
                                    c = h[2 * i + 1][j]
                                    d = h[2 * i + 2][j]
                                    v[i * 7 + j] = (a + d) + (b + c) * _THREE
                        if r == 13:
                            for j in range(7):
                                v[6 * 7 + j] = h[13][j] * _B0 + (
                                    h[12][j] * _B0 + h[11][j] * _B1
                                )
                    thr = _select_threshold(v)
                    mb = p * (_C * 16)
                    for q in range(_C):
                        mbuf[pl.ds(mb + q * 16, 16)] = (
                            v[q] < thr
                        ).astype(jnp.int32)
                    return carry3

                lax.fori_loop(0, 4, pass_body, 0)

                comb = []
                for q in range(_C):
                    w0 = mbuf[pl.ds(0 * (_C * 16) + q * 16, 16)]
                    w1 = mbuf[pl.ds(1 * (_C * 16) + q * 16, 16)]
                    w2 = mbuf[pl.ds(2 * (_C * 16) + q * 16, 16)]
                    w3 = mbuf[pl.ds(3 * (_C * 16) + q * 16, 16)]
                    comb.append(
                        w0 + w1 * jnp.int32(1 << 8)
                        + w2 * jnp.int32(1 << 16) + w3 * jnp.int32(1 << 24)
                    )
                wrow = g * 16 + iota
                for f in range(_F):
                    plsc.store_scatter(
                        out_v,
                        [wrow, jnp.full((16,), f, jnp.int32)],
                        comb[_CELL[f]],
                    )
                return carry2

            lax.fori_loop(0, _NG, group_body, 0)
            obase = pl.multiple_of(base // 4, _CHUNK // 4)
            pltpu.sync_copy(out_v, out_hbm.at[pl.ds(obase, _CHUNK // 4)])
            return carry

        lax.fori_loop(0, _NCHUNK, chunk_body, 0)

    return body(att)


def kernel(attention):
    assert attention.shape == (_BS, _F) and attention.dtype == jnp.float32
    return _sc_mask_bytes(attention.reshape(_BS * _F)) != 0
